# Initial kernel scaffold; baseline (speedup 1.0000x reference)
import jax
import jax.numpy as jnp
from jax.experimental import pallas as pl


def kernel(output, target, token_type):
    raise NotImplementedError("write your pallas kernel here")


# speedup vs baseline: 1.9468x; 1.9468x over previous
import jax
import jax.numpy as jnp
from jax.experimental import pallas as pl
from jax.experimental.pallas import tpu as pltpu

_WEIGHT = 1.0


def _loss_body(x_ref, tgt_ref, tt_ref, nll_ref, msk_ref):
    x = x_ref[0]
    tgt = tgt_ref[0]
    tt = tt_ref[...]
    C, S = x.shape

    m_r = jnp.max(x, axis=1, keepdims=True)
    lse_r = m_r + jnp.log(jnp.sum(jnp.exp(x - m_r), axis=1, keepdims=True))
    score = x - lse_r

    m_c = jnp.max(x, axis=0, keepdims=True)
    lse_c = m_c + jnp.log(jnp.sum(jnp.exp(x - m_c), axis=0, keepdims=True))

    c_iota = jax.lax.broadcasted_iota(jnp.int32, (C, S), 0)
    is_tgt = c_iota == tgt
    x_tgt = jnp.sum(jnp.where(is_tgt, x, 0.0), axis=0, keepdims=True)
    tt_tgt = jnp.sum(jnp.where(is_tgt, tt, 0), axis=0, keepdims=True)

    s_max = jnp.max(score, axis=0, keepdims=True)
    packed = jnp.where(score == s_max, c_iota * 8 + tt, jnp.int32(C * 8))
    tt_pred = jnp.bitwise_and(jnp.min(packed, axis=0, keepdims=True), 7)

    nll_sum = jnp.sum(lse_c - x_tgt)
    msk_sum = jnp.sum((tt_pred != tt_tgt).astype(jnp.float32))
    nll_ref[0] = jnp.full((1, 128), nll_sum, dtype=jnp.float32)
    msk_ref[0] = jnp.full((1, 128), msk_sum, dtype=jnp.float32)


def kernel(output, target, token_type):
    B, C, S = output.shape
    tgt = target.astype(jnp.int32).reshape(B, 1, S)
    tt = token_type.astype(jnp.int32).reshape(C, 1)

    nll, msk = pl.pallas_call(
        _loss_body,
        grid=(B,),
        in_specs=[
            pl.BlockSpec((1, C, S), lambda b: (b, 0, 0)),
            pl.BlockSpec((1, 1, S), lambda b: (b, 0, 0)),
            pl.BlockSpec((C, 1), lambda b: (0, 0)),
        ],
        out_specs=(
            pl.BlockSpec((1, 1, 128), lambda b: (b, 0, 0)),
            pl.BlockSpec((1, 1, 128), lambda b: (b, 0, 0)),
        ),
        out_shape=(
            jax.ShapeDtypeStruct((B, 1, 128), jnp.float32),
            jax.ShapeDtypeStruct((B, 1, 128), jnp.float32),
        ),
        compiler_params=pltpu.CompilerParams(
            dimension_semantics=("parallel",),
        ),
    )(output, tgt, tt)

    denom = jnp.float32(B * S)
    loss = jnp.sum(nll[:, 0, 0]) / denom
    mask_mean = jnp.sum(msk[:, 0, 0]) / denom
    return loss + _WEIGHT * loss * mask_mean
